# Initial kernel scaffold; baseline (speedup 1.0000x reference)
import jax
import jax.numpy as jnp
from jax.experimental import pallas as pl


def kernel(pos, vel, dt):
    raise NotImplementedError("write your pallas kernel here")


# speedup vs baseline: 4080.9252x; 4080.9252x over previous
import functools

import jax
import jax.numpy as jnp
from jax.experimental import pallas as pl

_H = 0.3
_DIM = 2
_RHO0 = 1000.0
_C0 = 10.0
_NU = 0.0001
_GAMMA = 7.0
_B = _RHO0 * _C0 ** 2 / _GAMMA
_PI = 3.14159265
_SIGMA = 10.0 / (7.0 * _PI)
_ALPHA = _SIGMA / _H ** _DIM
_INV_H = 1.0 / _H
_GCOEF = _ALPHA / _H ** 2
_GRAV_Y = -9.81

_BR = 256
_BC = 1024


def _pressure_from_rho(rho):
    x = rho * (1.0 / _RHO0)
    x2 = x * x
    x3 = x2 * x
    return _B * (x3 * x3 * x - 1.0)


def _density_body(pos_ref, posT_ref, rho_ref):
    c = pl.program_id(1)
    nc = pl.num_programs(1)
    x_i = pos_ref[:, 0:1]
    y_i = pos_ref[:, 1:2]
    x_j = posT_ref[0:1, :]
    y_j = posT_ref[1:2, :]
    dx = x_j - x_i
    dy = y_j - y_i
    d2 = dx * dx + dy * dy
    dist = jnp.sqrt(jnp.maximum(d2, 1e-24))
    q = jnp.minimum(dist * _INV_H, 2.0)
    w_in = 1.0 - 1.5 * q * q + 0.75 * q * q * q
    t = 2.0 - q
    w_out = 0.25 * t * t * t
    w = _ALPHA * jnp.where(q < 1.0, w_in, w_out)
    part = jnp.sum(w, axis=1, keepdims=True)

    @pl.when(c == 0)
    def _():
        rho_ref[...] = part

    @pl.when(c > 0)
    def _():
        rho_ref[...] = rho_ref[...] + part

    @pl.when(c == nc - 1)
    def _():
        rho_ref[...] = jnp.maximum(rho_ref[...], 0.0001)


def _force_body(pos_ref, vel_ref, rho_i_ref, posT_ref, velT_ref, rho_j_ref,
                dt_ref, pos_out_ref, vel_out_ref):
    c = pl.program_id(1)
    nc = pl.num_programs(1)
    x_i = pos_ref[:, 0:1]
    y_i = pos_ref[:, 1:2]
    x_j = posT_ref[0:1, :]
    y_j = posT_ref[1:2, :]
    dx = x_j - x_i
    dy = y_j - y_i
    d2 = dx * dx + dy * dy
    dist = jnp.sqrt(jnp.maximum(d2, 1e-24))
    mask = (dist < _H) & (dist > 1e-10)
    q = dist * _INV_H
    cgrad = _GCOEF * (2.25 * q - 3.0)

    rho_i = rho_i_ref[...]
    rho_j = rho_j_ref[...]
    p_i = _pressure_from_rho(rho_i)
    p_j = _pressure_from_rho(rho_j)
    p_term = p_i / (rho_i * rho_i) + p_j / (rho_j * rho_j)
    pref = -p_term * cgrad

    lap = (2.0 * _NU) * cgrad * (d2 / (rho_j * jnp.maximum(d2, 1e-10)))
    dvx = velT_ref[0:1, :] - vel_ref[:, 0:1]
    dvy = velT_ref[1:2, :] - vel_ref[:, 1:2]

    fx = jnp.where(mask, pref * dx + dvx * lap, 0.0)
    fy = jnp.where(mask, pref * dy + dvy * lap, 0.0)
    fsum = jnp.concatenate(
        [jnp.sum(fx, axis=1, keepdims=True), jnp.sum(fy, axis=1, keepdims=True)],
        axis=1)

    @pl.when(c == 0)
    def _():
        vel_out_ref[...] = fsum

    @pl.when(c > 0)
    def _():
        vel_out_ref[...] = vel_out_ref[...] + fsum

    @pl.when(c == nc - 1)
    def _():
        dt_v = dt_ref[0, 0]
        new_vx = vel_ref[:, 0:1] + dt_v * vel_out_ref[:, 0:1]
        new_vy = vel_ref[:, 1:2] + dt_v * (vel_out_ref[:, 1:2] + _GRAV_Y)
        new_vel = jnp.concatenate([new_vx, new_vy], axis=1)
        vel_out_ref[...] = new_vel
        pos_out_ref[...] = pos_ref[...] + dt_v * new_vel


@functools.partial(jax.jit, static_argnums=())
def kernel(pos, vel, dt):
    n = pos.shape[0]
    pos = pos.astype(jnp.float32)
    vel = vel.astype(jnp.float32)
    pos_t = pos.T
    vel_t = vel.T
    dt_arr = jnp.asarray(dt, jnp.float32).reshape(1, 1)
    nr = n // _BR
    nc = n // _BC

    rho = pl.pallas_call(
        _density_body,
        grid=(nr, nc),
        in_specs=[
            pl.BlockSpec((_BR, _DIM), lambda r, c: (r, 0)),
            pl.BlockSpec((_DIM, _BC), lambda r, c: (0, c)),
        ],
        out_specs=pl.BlockSpec((_BR, 1), lambda r, c: (r, 0)),
        out_shape=jax.ShapeDtypeStruct((n, 1), jnp.float32),
    )(pos, pos_t)

    rho_row = rho.reshape(1, n)

    new_pos, new_vel = pl.pallas_call(
        _force_body,
        grid=(nr, nc),
        in_specs=[
            pl.BlockSpec((_BR, _DIM), lambda r, c: (r, 0)),
            pl.BlockSpec((_BR, _DIM), lambda r, c: (r, 0)),
            pl.BlockSpec((_BR, 1), lambda r, c: (r, 0)),
            pl.BlockSpec((_DIM, _BC), lambda r, c: (0, c)),
            pl.BlockSpec((_DIM, _BC), lambda r, c: (0, c)),
            pl.BlockSpec((1, _BC), lambda r, c: (0, c)),
            pl.BlockSpec((1, 1), lambda r, c: (0, 0)),
        ],
        out_specs=[
            pl.BlockSpec((_BR, _DIM), lambda r, c: (r, 0)),
            pl.BlockSpec((_BR, _DIM), lambda r, c: (r, 0)),
        ],
        out_shape=[
            jax.ShapeDtypeStruct((n, _DIM), jnp.float32),
            jax.ShapeDtypeStruct((n, _DIM), jnp.float32),
        ],
    )(pos, vel, rho, pos_t, vel_t, rho_row, dt_arr)

    return (new_pos, new_vel)


# speedup vs baseline: 4798.5788x; 1.1759x over previous
import functools

import jax
import jax.numpy as jnp
from jax.experimental import pallas as pl

_H = 0.3
_DIM = 2
_RHO0 = 1000.0
_C0 = 10.0
_NU = 0.0001
_GAMMA = 7.0
_B = _RHO0 * _C0 ** 2 / _GAMMA
_PI = 3.14159265
_SIGMA = 10.0 / (7.0 * _PI)
_ALPHA = _SIGMA / _H ** _DIM
_INV_H = 1.0 / _H
_GCOEF = _ALPHA / _H ** 2
_GRAV_Y = -9.81

_BR = 256
_BC = 4096
_CG_A = 2.25 * _GCOEF * _INV_H
_CG_B = -3.0 * _GCOEF


def _pressure_from_rho(rho):
    x = rho * (1.0 / _RHO0)
    x2 = x * x
    x3 = x2 * x
    return _B * (x3 * x3 * x - 1.0)


def _density_body(pos_ref, posT_ref, rho_ref):
    c = pl.program_id(1)
    nc = pl.num_programs(1)
    x_i = pos_ref[:, 0:1]
    y_i = pos_ref[:, 1:2]
    x_j = posT_ref[0:1, :]
    y_j = posT_ref[1:2, :]
    dx = x_j - x_i
    dy = y_j - y_i
    d2 = dx * dx + dy * dy
    dist = jnp.sqrt(jnp.maximum(d2, 1e-24))
    q = jnp.minimum(dist * _INV_H, 2.0)
    q2 = q * q
    w_in = _ALPHA + q2 * ((0.75 * _ALPHA) * q - (1.5 * _ALPHA))
    t = 2.0 - q
    w_out = (0.25 * _ALPHA) * (t * t) * t
    w = jnp.where(q < 1.0, w_in, w_out)
    part = jnp.sum(w, axis=1, keepdims=True)

    @pl.when(c == 0)
    def _():
        rho_ref[...] = part

    @pl.when(c > 0)
    def _():
        rho_ref[...] = rho_ref[...] + part

    @pl.when(c == nc - 1)
    def _():
        rho_ref[...] = jnp.maximum(rho_ref[...], 0.0001)


def _force_body(pos_ref, vel_ref, rho_i_ref, posT_ref, velT_ref, rho_j_ref,
                dt_ref, pos_out_ref, vel_out_ref):
    c = pl.program_id(1)
    nc = pl.num_programs(1)
    x_i = pos_ref[:, 0:1]
    y_i = pos_ref[:, 1:2]
    x_j = posT_ref[0:1, :]
    y_j = posT_ref[1:2, :]
    dx = x_j - x_i
    dy = y_j - y_i
    d2 = dx * dx + dy * dy
    dist = jnp.sqrt(jnp.maximum(d2, 1e-24))
    mask = (dist < _H) & (dist > 1e-10)
    cgrad = _CG_A * dist + _CG_B

    rho_i = rho_i_ref[...]
    rho_j = rho_j_ref[...]
    p_i = _pressure_from_rho(rho_i)
    p_j = _pressure_from_rho(rho_j)
    npi_term = -(p_i / (rho_i * rho_i))
    npj_term = -(p_j / (rho_j * rho_j))
    pref = (npi_term + npj_term) * cgrad

    vcoef_j = (2.0 * _NU) / rho_j
    visc = cgrad * jnp.minimum(d2 * 1e10, 1.0) * vcoef_j
    dvx = velT_ref[0:1, :] - vel_ref[:, 0:1]
    dvy = velT_ref[1:2, :] - vel_ref[:, 1:2]

    fx = jnp.where(mask, pref * dx + dvx * visc, 0.0)
    fy = jnp.where(mask, pref * dy + dvy * visc, 0.0)
    fsum = jnp.concatenate(
        [jnp.sum(fx, axis=1, keepdims=True), jnp.sum(fy, axis=1, keepdims=True)],
        axis=1)

    @pl.when(c == 0)
    def _():
        vel_out_ref[...] = fsum

    @pl.when(c > 0)
    def _():
        vel_out_ref[...] = vel_out_ref[...] + fsum

    @pl.when(c == nc - 1)
    def _():
        dt_v = dt_ref[0, 0]
        new_vx = vel_ref[:, 0:1] + dt_v * vel_out_ref[:, 0:1]
        new_vy = vel_ref[:, 1:2] + dt_v * (vel_out_ref[:, 1:2] + _GRAV_Y)
        new_vel = jnp.concatenate([new_vx, new_vy], axis=1)
        vel_out_ref[...] = new_vel
        pos_out_ref[...] = pos_ref[...] + dt_v * new_vel


@functools.partial(jax.jit, static_argnums=())
def kernel(pos, vel, dt):
    n = pos.shape[0]
    pos = pos.astype(jnp.float32)
    vel = vel.astype(jnp.float32)
    pos_t = pos.T
    vel_t = vel.T
    dt_arr = jnp.asarray(dt, jnp.float32).reshape(1, 1)
    nr = n // _BR
    nc = n // _BC

    rho = pl.pallas_call(
        _density_body,
        grid=(nr, nc),
        in_specs=[
            pl.BlockSpec((_BR, _DIM), lambda r, c: (r, 0)),
            pl.BlockSpec((_DIM, _BC), lambda r, c: (0, c)),
        ],
        out_specs=pl.BlockSpec((_BR, 1), lambda r, c: (r, 0)),
        out_shape=jax.ShapeDtypeStruct((n, 1), jnp.float32),
    )(pos, pos_t)

    rho_row = rho.reshape(1, n)

    new_pos, new_vel = pl.pallas_call(
        _force_body,
        grid=(nr, nc),
        in_specs=[
            pl.BlockSpec((_BR, _DIM), lambda r, c: (r, 0)),
            pl.BlockSpec((_BR, _DIM), lambda r, c: (r, 0)),
            pl.BlockSpec((_BR, 1), lambda r, c: (r, 0)),
            pl.BlockSpec((_DIM, _BC), lambda r, c: (0, c)),
            pl.BlockSpec((_DIM, _BC), lambda r, c: (0, c)),
            pl.BlockSpec((1, _BC), lambda r, c: (0, c)),
            pl.BlockSpec((1, 1), lambda r, c: (0, 0)),
        ],
        out_specs=[
            pl.BlockSpec((_BR, _DIM), lambda r, c: (r, 0)),
            pl.BlockSpec((_BR, _DIM), lambda r, c: (r, 0)),
        ],
        out_shape=[
            jax.ShapeDtypeStruct((n, _DIM), jnp.float32),
            jax.ShapeDtypeStruct((n, _DIM), jnp.float32),
        ],
    )(pos, vel, rho, pos_t, vel_t, rho_row, dt_arr)

    return (new_pos, new_vel)
